# baseline (speedup 1.0000x reference)
import functools

import jax
import jax.numpy as jnp
from jax import lax
from jax.experimental import pallas as pl
from jax.experimental.pallas import tpu as pltpu
from jax.experimental.pallas import tpu_sc as plsc

_B, _NB, _T, _D, _K = 4, 96, 256, 64, 8192
_KC = 2048


def _mm(a, b):
    return lax.dot_general(a, b, (((1,), (0,)), ((), ())),
                           preferred_element_type=jnp.float32)


def _mm_t(a, b):
    return lax.dot_general(a, b, (((1,), (1,)), ((), ())),
                           preferred_element_type=jnp.float32)


def _conv1d(x, w, b):
    y = lax.conv_general_dilated(x, w, window_strides=(1,), padding='SAME',
                                 dimension_numbers=('NCH', 'OIH', 'NCH'))
    return y + b[None, :, None]


def _attn_blk(x, a):
    xt = jnp.transpose(x, (0, 2, 1))
    q = xt @ a['wq'] + a['bq']
    k = xt @ a['wk'] + a['bk']
    v = xt @ a['wv'] + a['bv']
    scale = jnp.sqrt(jnp.asarray(q.shape[-1], dtype=x.dtype))
    attn = jax.nn.softmax(q @ jnp.transpose(k, (0, 2, 1)) / scale, axis=-1)
    o = (attn @ v) @ a['wo'] + a['bo']
    return x + jnp.transpose(o, (0, 2, 1))


def _encode(x, p):
    z = x * p['w_proj'][None, :, None]
    for w, b in p['enc_conv']:
        z = jax.nn.relu(_conv1d(z, w, b))
    for a in p['enc_attn']:
        z = _attn_blk(z, a)
    return z


def _vq_body(zt_ref, emb_ref, e2_ref, idx_ref):
    f = zt_ref[0]
    f2 = jnp.sum(f * f, axis=1, keepdims=True)
    ids = lax.broadcasted_iota(jnp.int32, (_T, _KC), 1)

    def chunk(j, carry):
        bd, bi = carry
        e = emb_ref[pl.ds(j * _KC, _KC), :]
        g = _mm_t(f, e)
        e2 = e2_ref[0, pl.ds(j * _KC, _KC)][None, :]
        d = (f2 - 2.0 * g) + e2
        dmin = jnp.min(d, axis=1, keepdims=True)
        imin = jnp.min(jnp.where(d == dmin, ids, jnp.int32(2**31 - 1)),
                       axis=1, keepdims=True) + j * _KC
        better = dmin < bd
        return jnp.where(better, dmin, bd), jnp.where(better, imin, bi)

    bd0 = jnp.full((_T, 1), jnp.inf, jnp.float32)
    bi0 = jnp.zeros((_T, 1), jnp.int32)
    _, bi = lax.fori_loop(0, _K // _KC, chunk, (bd0, bi0))
    idx_ref[0] = bi


def _gather_sc(table, idx):
    dp = 128
    tablep = jnp.pad(table, ((0, 0), (0, dp - table.shape[1])))
    info = plsc.get_sparse_core_info()
    nw = info.num_cores * info.num_subcores
    bt = idx.shape[0]
    bpw = bt // nw
    nc = info.num_cores
    mesh = plsc.VectorSubcoreMesh(core_axis_name="c", subcore_axis_name="s")

    @functools.partial(
        pl.kernel, mesh=mesh,
        out_type=jax.ShapeDtypeStruct((bt, dp), jnp.float32),
        scratch_types=[
            pltpu.VMEM((bpw,), jnp.int32),
            pltpu.VMEM((bpw, dp), jnp.float32),
            pltpu.SemaphoreType.DMA,
        ],
    )
    def k(table_hbm, idx_hbm, out_hbm, idx_v, rows_v, sem):
        wid = lax.axis_index("s") * nc + lax.axis_index("c")
        base = wid * bpw
        pltpu.sync_copy(idx_hbm.at[pl.ds(base, bpw)], idx_v)
        pltpu.async_copy(table_hbm.at[idx_v], rows_v, sem).wait()
        pltpu.sync_copy(rows_v, out_hbm.at[pl.ds(base, bpw)])

    return k(tablep, idx)[:, :_D]


def _conv3(h, w0, w1, w2, b):
    zrow = jnp.zeros((1, h.shape[1]), h.dtype)
    prev = jnp.concatenate([zrow, h[:-1]], axis=0)
    nxt = jnp.concatenate([h[1:], zrow], axis=0)
    y = _mm(prev, w0) + _mm(h, w1) + _mm(nxt, w2) + b
    return jnp.maximum(y, 0.0)


def _attn(h, wq, bq, wk, bk, wv, bv, wo, bo):
    q = _mm(h, wq) + bq
    k = _mm(h, wk) + bk
    v = _mm(h, wv) + bv
    s = _mm_t(q, k) * (1.0 / 8.0)
    m = jnp.max(s, axis=1, keepdims=True)
    e = jnp.exp(s - m)
    a = e / jnp.sum(e, axis=1, keepdims=True)
    o = _mm(_mm(a, v), wo) + bo
    return h + o


def _dec_body(idx_sref, emb_ref,
              a0wq, a0bq, a0wk, a0bk, a0wv, a0bv, a0wo, a0bo,
              a1wq, a1bq, a1wk, a1bk, a1wv, a1bv, a1wo, a1bo,
              d1w0, d1w1, d1w2, d1b,
              d2w0, d2w1, d2w2, d2b,
              d3w0, d3w1, d3w2, d3b,
              rw_ref, rb_ref,
              out_ref, zq_scr):
    base = pl.program_id(0) * _T

    def g(t, c):
        zq_scr[pl.ds(t, 1), :] = emb_ref[pl.ds(idx_sref[base + t], 1), :]
        return c

    lax.fori_loop(0, _T, g, 0)
    h = zq_scr[...]
    h = _attn(h, a0wq[...], a0bq[...], a0wk[...], a0bk[...],
              a0wv[...], a0bv[...], a0wo[...], a0bo[...])
    h = _attn(h, a1wq[...], a1bq[...], a1wk[...], a1bk[...],
              a1wv[...], a1bv[...], a1wo[...], a1bo[...])
    h = _conv3(h, d1w0[...], d1w1[...], d1w2[...], d1b[...])
    h = _conv3(h, d2w0[...], d2w1[...], d2w2[...], d2b[...])
    h = _conv3(h, d3w0[...], d3w1[...], d3w2[...], d3b[...])
    out_ref[0] = _mm(h, rw_ref[...]) + rb_ref[...]


def _full_spec(arr):
    nd = arr.ndim
    return pl.BlockSpec(arr.shape, lambda b, *_, _nd=nd: (0,) * _nd)


def _attn_flat(a):
    return [a['wq'], a['bq'].reshape(1, -1), a['wk'], a['bk'].reshape(1, -1),
            a['wv'], a['bv'].reshape(1, -1), a['wo'], a['bo'].reshape(1, -1)]


def _conv_flat(wb):
    w, b = wb
    return [w[:, :, 0].T, w[:, :, 1].T, w[:, :, 2].T, b.reshape(1, -1)]


def kernel(x, params):
    p = params
    emb = p['emb']

    z = _encode(x, p)
    zt = jnp.transpose(z, (0, 2, 1))
    e2 = jnp.sum(emb**2, axis=1)[None, :]

    idx3 = pl.pallas_call(
        _vq_body,
        grid=(_B,),
        in_specs=[pl.BlockSpec((1, _T, _D), lambda b: (b, 0, 0)),
                  _full_spec(emb), _full_spec(e2)],
        out_specs=pl.BlockSpec((1, _T, 1), lambda b: (b, 0, 0)),
        out_shape=jax.ShapeDtypeStruct((_B, _T, 1), jnp.int32),
    )(zt, emb, e2)

    indices = idx3.reshape(_B, _T)
    idx_flat = idx3.reshape(-1)

    zq_flat = _gather_sc(emb, idx_flat)
    zq = jnp.transpose(zq_flat.reshape(_B, _T, _D), (0, 2, 1))

    dec_in = [emb]
    for a in p['dec_attn']:
        dec_in += _attn_flat(a)
    for wb in p['dec_conv']:
        dec_in += _conv_flat(wb)
    dec_in += [p['rw'], p['rb'].reshape(1, -1)]

    recont = pl.pallas_call(
        _dec_body,
        grid_spec=pltpu.PrefetchScalarGridSpec(
            num_scalar_prefetch=1,
            grid=(_B,),
            in_specs=[_full_spec(a) for a in dec_in],
            out_specs=pl.BlockSpec((1, _T, _NB), lambda b, *_: (b, 0, 0)),
            scratch_shapes=[pltpu.VMEM((_T, _D), jnp.float32)],
        ),
        out_shape=jax.ShapeDtypeStruct((_B, _T, _NB), jnp.float32),
    )(idx_flat, *dec_in)

    recon = jnp.transpose(recont, (0, 2, 1))
    return recon, z, zq, indices


# speedup vs baseline: 1.3809x; 1.3809x over previous
import functools

import jax
import jax.numpy as jnp
from jax import lax
from jax.experimental import pallas as pl
from jax.experimental.pallas import tpu as pltpu
from jax.experimental.pallas import tpu_sc as plsc

_B, _NB, _T, _D, _K = 4, 96, 256, 64, 8192
_KC = 2048


def _mm(a, b):
    return lax.dot_general(a, b, (((1,), (0,)), ((), ())),
                           preferred_element_type=jnp.float32)


def _mm_t(a, b):
    return lax.dot_general(a, b, (((1,), (1,)), ((), ())),
                           preferred_element_type=jnp.float32)


def _conv1d(x, w, b):
    y = lax.conv_general_dilated(x, w, window_strides=(1,), padding='SAME',
                                 dimension_numbers=('NCH', 'OIH', 'NCH'))
    return y + b[None, :, None]


def _attn_blk(x, a):
    xt = jnp.transpose(x, (0, 2, 1))
    q = xt @ a['wq'] + a['bq']
    k = xt @ a['wk'] + a['bk']
    v = xt @ a['wv'] + a['bv']
    scale = jnp.sqrt(jnp.asarray(q.shape[-1], dtype=x.dtype))
    attn = jax.nn.softmax(q @ jnp.transpose(k, (0, 2, 1)) / scale, axis=-1)
    o = (attn @ v) @ a['wo'] + a['bo']
    return x + jnp.transpose(o, (0, 2, 1))


def _encode(x, p):
    z = x * p['w_proj'][None, :, None]
    for w, b in p['enc_conv']:
        z = jax.nn.relu(_conv1d(z, w, b))
    for a in p['enc_attn']:
        z = _attn_blk(z, a)
    return z


_NT = _B * _T


def _vq_body(zt_ref, emb_ref, e2_ref, idx_ref):
    f = zt_ref[...]
    f2 = jnp.sum(f * f, axis=1, keepdims=True)
    ids = lax.broadcasted_iota(jnp.int32, (_NT, _KC), 1)

    def chunk(j, carry):
        bd, bi = carry
        e = emb_ref[pl.ds(j * _KC, _KC), :]
        g = _mm_t(f, e)
        e2 = e2_ref[0, pl.ds(j * _KC, _KC)][None, :]
        d = (f2 - 2.0 * g) + e2
        dmin = jnp.min(d, axis=1, keepdims=True)
        imin = jnp.min(jnp.where(d == dmin, ids, jnp.int32(2**31 - 1)),
                       axis=1, keepdims=True) + j * _KC
        better = dmin < bd
        return jnp.where(better, dmin, bd), jnp.where(better, imin, bi)

    bd0 = jnp.full((_NT, 1), jnp.inf, jnp.float32)
    bi0 = jnp.zeros((_NT, 1), jnp.int32)
    _, bi = lax.fori_loop(0, _K // _KC, chunk, (bd0, bi0))
    idx_ref[...] = bi


def _gather_sc(table, idx):
    dp = 128
    tablep = jnp.pad(table, ((0, 0), (0, dp - table.shape[1])))
    info = plsc.get_sparse_core_info()
    nw = info.num_cores * info.num_subcores
    bt = idx.shape[0]
    bpw = bt // nw
    nc = info.num_cores
    mesh = plsc.VectorSubcoreMesh(core_axis_name="c", subcore_axis_name="s")

    @functools.partial(
        pl.kernel, mesh=mesh,
        out_type=jax.ShapeDtypeStruct((bt, dp), jnp.float32),
        scratch_types=[
            pltpu.VMEM((bpw,), jnp.int32),
            pltpu.VMEM((bpw, dp), jnp.float32),
            pltpu.SemaphoreType.DMA,
        ],
    )
    def k(table_hbm, idx_hbm, out_hbm, idx_v, rows_v, sem):
        wid = lax.axis_index("s") * nc + lax.axis_index("c")
        base = wid * bpw
        pltpu.sync_copy(idx_hbm.at[pl.ds(base, bpw)], idx_v)
        pltpu.async_copy(table_hbm.at[idx_v], rows_v, sem).wait()
        pltpu.sync_copy(rows_v, out_hbm.at[pl.ds(base, bpw)])

    return k(tablep, idx)[:, :_D]


def _conv3(h, w0, w1, w2, b):
    zrow = jnp.zeros((1, h.shape[1]), h.dtype)
    prev = jnp.concatenate([zrow, h[:-1]], axis=0)
    nxt = jnp.concatenate([h[1:], zrow], axis=0)
    y = _mm(prev, w0) + _mm(h, w1) + _mm(nxt, w2) + b
    return jnp.maximum(y, 0.0)


def _attn(h, wq, bq, wk, bk, wv, bv, wo, bo):
    q = _mm(h, wq) + bq
    k = _mm(h, wk) + bk
    v = _mm(h, wv) + bv
    s = _mm_t(q, k) * (1.0 / 8.0)
    m = jnp.max(s, axis=1, keepdims=True)
    e = jnp.exp(s - m)
    a = e / jnp.sum(e, axis=1, keepdims=True)
    o = _mm(_mm(a, v), wo) + bo
    return h + o


def _dec_body(idx_sref, emb_ref,
              a0wq, a0bq, a0wk, a0bk, a0wv, a0bv, a0wo, a0bo,
              a1wq, a1bq, a1wk, a1bk, a1wv, a1bv, a1wo, a1bo,
              d1w0, d1w1, d1w2, d1b,
              d2w0, d2w1, d2w2, d2b,
              d3w0, d3w1, d3w2, d3b,
              rw_ref, rb_ref,
              out_ref, zqt_ref, zq_scr):
    base = pl.program_id(0) * _T

    def g(t, c):
        zq_scr[pl.ds(t, 1), :] = emb_ref[pl.ds(idx_sref[base + t], 1), :]
        return c

    lax.fori_loop(0, _T, g, 0)
    h = zq_scr[...]
    zqt_ref[0] = h
    h = _attn(h, a0wq[...], a0bq[...], a0wk[...], a0bk[...],
              a0wv[...], a0bv[...], a0wo[...], a0bo[...])
    h = _attn(h, a1wq[...], a1bq[...], a1wk[...], a1bk[...],
              a1wv[...], a1bv[...], a1wo[...], a1bo[...])
    h = _conv3(h, d1w0[...], d1w1[...], d1w2[...], d1b[...])
    h = _conv3(h, d2w0[...], d2w1[...], d2w2[...], d2b[...])
    h = _conv3(h, d3w0[...], d3w1[...], d3w2[...], d3b[...])
    out_ref[0] = _mm(h, rw_ref[...]) + rb_ref[...]


def _full_spec(arr):
    nd = arr.ndim
    return pl.BlockSpec(arr.shape, lambda b, *_, _nd=nd: (0,) * _nd)


def _attn_flat(a):
    return [a['wq'], a['bq'].reshape(1, -1), a['wk'], a['bk'].reshape(1, -1),
            a['wv'], a['bv'].reshape(1, -1), a['wo'], a['bo'].reshape(1, -1)]


def _conv_flat(wb):
    w, b = wb
    return [w[:, :, 0].T, w[:, :, 1].T, w[:, :, 2].T, b.reshape(1, -1)]


def kernel(x, params):
    p = params
    emb = p['emb']

    z = _encode(x, p)
    zt = jnp.transpose(z, (0, 2, 1))
    e2 = jnp.sum(emb**2, axis=1)[None, :]

    idx2 = pl.pallas_call(
        _vq_body,
        grid=(1,),
        in_specs=[pl.BlockSpec((_NT, _D), lambda b, *_: (0, 0)),
                  _full_spec(emb), _full_spec(e2)],
        out_specs=pl.BlockSpec((_NT, 1), lambda b, *_: (0, 0)),
        out_shape=jax.ShapeDtypeStruct((_NT, 1), jnp.int32),
    )(zt.reshape(_NT, _D), emb, e2)

    indices = idx2.reshape(_B, _T)
    idx_flat = idx2.reshape(-1)

    dec_in = [emb]
    for a in p['dec_attn']:
        dec_in += _attn_flat(a)
    for wb in p['dec_conv']:
        dec_in += _conv_flat(wb)
    dec_in += [p['rw'], p['rb'].reshape(1, -1)]

    recont, zqt = pl.pallas_call(
        _dec_body,
        grid_spec=pltpu.PrefetchScalarGridSpec(
            num_scalar_prefetch=1,
            grid=(_B,),
            in_specs=[_full_spec(a) for a in dec_in],
            out_specs=[pl.BlockSpec((1, _T, _NB), lambda b, *_: (b, 0, 0)),
                       pl.BlockSpec((1, _T, _D), lambda b, *_: (b, 0, 0))],
            scratch_shapes=[pltpu.VMEM((_T, _D), jnp.float32)],
        ),
        out_shape=[jax.ShapeDtypeStruct((_B, _T, _NB), jnp.float32),
                   jax.ShapeDtypeStruct((_B, _T, _D), jnp.float32)],
    )(idx_flat, *dec_in)

    recon = jnp.transpose(recont, (0, 2, 1))
    zq = jnp.transpose(zqt, (0, 2, 1))
    return recon, z, zq, indices


# speedup vs baseline: 1.4544x; 1.0533x over previous
import functools

import jax
import jax.numpy as jnp
from jax import lax
from jax.experimental import pallas as pl
from jax.experimental.pallas import tpu as pltpu
from jax.experimental.pallas import tpu_sc as plsc

_B, _NB, _T, _D, _K = 4, 96, 256, 64, 8192
_KC = 2048


def _mm(a, b):
    return lax.dot_general(a, b, (((1,), (0,)), ((), ())),
                           preferred_element_type=jnp.float32)


def _mm_t(a, b):
    return lax.dot_general(a, b, (((1,), (1,)), ((), ())),
                           preferred_element_type=jnp.float32)


def _conv1d(x, w, b):
    y = lax.conv_general_dilated(x, w, window_strides=(1,), padding='SAME',
                                 dimension_numbers=('NCH', 'OIH', 'NCH'))
    return y + b[None, :, None]


def _attn_blk(x, a):
    xt = jnp.transpose(x, (0, 2, 1))
    q = xt @ a['wq'] + a['bq']
    k = xt @ a['wk'] + a['bk']
    v = xt @ a['wv'] + a['bv']
    scale = jnp.sqrt(jnp.asarray(q.shape[-1], dtype=x.dtype))
    attn = jax.nn.softmax(q @ jnp.transpose(k, (0, 2, 1)) / scale, axis=-1)
    o = (attn @ v) @ a['wo'] + a['bo']
    return x + jnp.transpose(o, (0, 2, 1))


def _encode(x, p):
    z = x * p['w_proj'][None, :, None]
    for w, b in p['enc_conv']:
        z = jax.nn.relu(_conv1d(z, w, b))
    for a in p['enc_attn']:
        z = _attn_blk(z, a)
    return z


_NT = _B * _T


def _vq_body(zt_ref, emb_ref, e2_ref, idx_ref):
    f = zt_ref[...]
    f2 = jnp.sum(f * f, axis=1, keepdims=True)
    ids = lax.broadcasted_iota(jnp.int32, (_NT, _KC), 1)

    def chunk(j, carry):
        bd, bi = carry
        e = emb_ref[pl.ds(j * _KC, _KC), :]
        g2 = _mm_t(f, e * -2.0)
        e2 = e2_ref[0, pl.ds(j * _KC, _KC)][None, :]
        d = (f2 + g2) + e2
        dmin = jnp.min(d, axis=1, keepdims=True)
        imin = jnp.min(jnp.where(d == dmin, ids, jnp.int32(2**31 - 1)),
                       axis=1, keepdims=True) + j * _KC
        better = dmin < bd
        return jnp.where(better, dmin, bd), jnp.where(better, imin, bi)

    bd0 = jnp.full((_NT, 1), jnp.inf, jnp.float32)
    bi0 = jnp.zeros((_NT, 1), jnp.int32)
    _, bi = lax.fori_loop(0, _K // _KC, chunk, (bd0, bi0))
    idx_ref[...] = bi


def _gather_sc(table, idx):
    dp = 128
    tablep = jnp.pad(table, ((0, 0), (0, dp - table.shape[1])))
    info = plsc.get_sparse_core_info()
    nw = info.num_cores * info.num_subcores
    bt = idx.shape[0]
    bpw = bt // nw
    nc = info.num_cores
    mesh = plsc.VectorSubcoreMesh(core_axis_name="c", subcore_axis_name="s")

    @functools.partial(
        pl.kernel, mesh=mesh,
        out_type=jax.ShapeDtypeStruct((bt, dp), jnp.float32),
        scratch_types=[
            pltpu.VMEM((bpw,), jnp.int32),
            pltpu.VMEM((bpw, dp), jnp.float32),
            pltpu.SemaphoreType.DMA,
        ],
    )
    def k(table_hbm, idx_hbm, out_hbm, idx_v, rows_v, sem):
        wid = lax.axis_index("s") * nc + lax.axis_index("c")
        base = wid * bpw
        pltpu.sync_copy(idx_hbm.at[pl.ds(base, bpw)], idx_v)
        pltpu.async_copy(table_hbm.at[idx_v], rows_v, sem).wait()
        pltpu.sync_copy(rows_v, out_hbm.at[pl.ds(base, bpw)])

    return k(tablep, idx)[:, :_D]


def _conv3(h, w0, w1, w2, b):
    zrow = jnp.zeros((1, h.shape[1]), h.dtype)
    prev = jnp.concatenate([zrow, h[:-1]], axis=0)
    nxt = jnp.concatenate([h[1:], zrow], axis=0)
    y = _mm(prev, w0) + _mm(h, w1) + _mm(nxt, w2) + b
    return jnp.maximum(y, 0.0)


def _attn(h, wq, bq, wk, bk, wv, bv, wo, bo):
    q = _mm(h, wq) + bq
    k = _mm(h, wk) + bk
    v = _mm(h, wv) + bv
    s = _mm_t(q, k) * (1.0 / 8.0)
    m = jnp.max(s, axis=1, keepdims=True)
    e = jnp.exp(s - m)
    a = e / jnp.sum(e, axis=1, keepdims=True)
    o = _mm(_mm(a, v), wo) + bo
    return h + o


def _dec_body(idx_sref, emb_ref,
              a0wq, a0bq, a0wk, a0bk, a0wv, a0bv, a0wo, a0bo,
              a1wq, a1bq, a1wk, a1bk, a1wv, a1bv, a1wo, a1bo,
              d1w0, d1w1, d1w2, d1b,
              d2w0, d2w1, d2w2, d2b,
              d3w0, d3w1, d3w2, d3b,
              rw_ref, rb_ref,
              out_ref, zqt_ref, zq_scr):
    base = pl.program_id(0) * _T

    def g(i, c):
        t0 = i * 8
        rows = [emb_ref[pl.ds(idx_sref[base + t0 + u], 1), :] for u in range(8)]
        zq_scr[pl.ds(t0, 8), :] = jnp.concatenate(rows, axis=0)
        return c

    lax.fori_loop(0, _T // 8, g, 0)
    h = zq_scr[...]
    zqt_ref[0] = h
    h = _attn(h, a0wq[...], a0bq[...], a0wk[...], a0bk[...],
              a0wv[...], a0bv[...], a0wo[...], a0bo[...])
    h = _attn(h, a1wq[...], a1bq[...], a1wk[...], a1bk[...],
              a1wv[...], a1bv[...], a1wo[...], a1bo[...])
    h = _conv3(h, d1w0[...], d1w1[...], d1w2[...], d1b[...])
    h = _conv3(h, d2w0[...], d2w1[...], d2w2[...], d2b[...])
    h = _conv3(h, d3w0[...], d3w1[...], d3w2[...], d3b[...])
    out_ref[0] = _mm(h, rw_ref[...]) + rb_ref[...]


def _full_spec(arr):
    nd = arr.ndim
    return pl.BlockSpec(arr.shape, lambda b, *_, _nd=nd: (0,) * _nd)


def _attn_flat(a):
    return [a['wq'], a['bq'].reshape(1, -1), a['wk'], a['bk'].reshape(1, -1),
            a['wv'], a['bv'].reshape(1, -1), a['wo'], a['bo'].reshape(1, -1)]


def _conv_flat(wb):
    w, b = wb
    return [w[:, :, 0].T, w[:, :, 1].T, w[:, :, 2].T, b.reshape(1, -1)]


def kernel(x, params):
    p = params
    emb = p['emb']

    z = _encode(x, p)
    zt = jnp.transpose(z, (0, 2, 1))
    e2 = jnp.sum(emb**2, axis=1)[None, :]

    idx2 = pl.pallas_call(
        _vq_body,
        grid=(1,),
        in_specs=[pl.BlockSpec((_NT, _D), lambda b, *_: (0, 0)),
                  _full_spec(emb), _full_spec(e2)],
        out_specs=pl.BlockSpec((_NT, 1), lambda b, *_: (0, 0)),
        out_shape=jax.ShapeDtypeStruct((_NT, 1), jnp.int32),
    )(zt.reshape(_NT, _D), emb, e2)

    indices = idx2.reshape(_B, _T)
    idx_flat = idx2.reshape(-1)

    dec_in = [emb]
    for a in p['dec_attn']:
        dec_in += _attn_flat(a)
    for wb in p['dec_conv']:
        dec_in += _conv_flat(wb)
    dec_in += [p['rw'], p['rb'].reshape(1, -1)]

    recont, zqt = pl.pallas_call(
        _dec_body,
        grid_spec=pltpu.PrefetchScalarGridSpec(
            num_scalar_prefetch=1,
            grid=(_B,),
            in_specs=[_full_spec(a) for a in dec_in],
            out_specs=[pl.BlockSpec((1, _T, _NB), lambda b, *_: (b, 0, 0)),
                       pl.BlockSpec((1, _T, _D), lambda b, *_: (b, 0, 0))],
            scratch_shapes=[pltpu.VMEM((_T, _D), jnp.float32)],
        ),
        out_shape=[jax.ShapeDtypeStruct((_B, _T, _NB), jnp.float32),
                   jax.ShapeDtypeStruct((_B, _T, _D), jnp.float32)],
    )(idx_flat, *dec_in)

    recon = jnp.transpose(recont, (0, 2, 1))
    zq = jnp.transpose(zqt, (0, 2, 1))
    return recon, z, zq, indices


# speedup vs baseline: 1.5459x; 1.0629x over previous
import functools

import jax
import jax.numpy as jnp
from jax import lax
from jax.experimental import pallas as pl
from jax.experimental.pallas import tpu as pltpu
from jax.experimental.pallas import tpu_sc as plsc

_B, _NB, _T, _D, _K = 4, 96, 256, 64, 8192
_KC = 2048


def _mm(a, b):
    return lax.dot_general(a, b, (((1,), (0,)), ((), ())),
                           preferred_element_type=jnp.float32)


def _mm_t(a, b):
    return lax.dot_general(a, b, (((1,), (1,)), ((), ())),
                           preferred_element_type=jnp.float32)


def _conv1d(x, w, b):
    y = lax.conv_general_dilated(x, w, window_strides=(1,), padding='SAME',
                                 dimension_numbers=('NCH', 'OIH', 'NCH'))
    return y + b[None, :, None]


def _attn_blk(x, a):
    xt = jnp.transpose(x, (0, 2, 1))
    q = xt @ a['wq'] + a['bq']
    k = xt @ a['wk'] + a['bk']
    v = xt @ a['wv'] + a['bv']
    scale = jnp.sqrt(jnp.asarray(q.shape[-1], dtype=x.dtype))
    attn = jax.nn.softmax(q @ jnp.transpose(k, (0, 2, 1)) / scale, axis=-1)
    o = (attn @ v) @ a['wo'] + a['bo']
    return x + jnp.transpose(o, (0, 2, 1))


def _encode(x, p):
    z = x * p['w_proj'][None, :, None]
    for w, b in p['enc_conv']:
        z = jax.nn.relu(_conv1d(z, w, b))
    for a in p['enc_attn']:
        z = _attn_blk(z, a)
    return z


_NT = _B * _T


def _vq_body(zt_ref, emb_ref, e2_ref, idx_ref):
    f = zt_ref[...]
    f2 = jnp.sum(f * f, axis=1, keepdims=True)
    ids = lax.broadcasted_iota(jnp.int32, (_NT, _KC), 1)

    def chunk(j, carry):
        bd, bi = carry
        e = emb_ref[pl.ds(j * _KC, _KC), :]
        g2 = _mm_t(f, e * -2.0)
        e2 = e2_ref[0, pl.ds(j * _KC, _KC)][None, :]
        d = (f2 + g2) + e2
        dmin = jnp.min(d, axis=1, keepdims=True)
        imin = jnp.min(jnp.where(d == dmin, ids, jnp.int32(2**31 - 1)),
                       axis=1, keepdims=True) + j * _KC
        better = dmin < bd
        return jnp.where(better, dmin, bd), jnp.where(better, imin, bi)

    bd0 = jnp.full((_NT, 1), jnp.inf, jnp.float32)
    bi0 = jnp.zeros((_NT, 1), jnp.int32)
    _, bi = lax.fori_loop(0, _K // _KC, chunk, (bd0, bi0))
    idx_ref[...] = bi


def _gather_sc(table, idx):
    dp = 128
    tablep = jnp.pad(table, ((0, 0), (0, dp - table.shape[1])))
    info = plsc.get_sparse_core_info()
    nw = info.num_cores * info.num_subcores
    bt = idx.shape[0]
    bpw = bt // nw
    nc = info.num_cores
    mesh = plsc.VectorSubcoreMesh(core_axis_name="c", subcore_axis_name="s")

    @functools.partial(
        pl.kernel, mesh=mesh,
        out_type=jax.ShapeDtypeStruct((bt, dp), jnp.float32),
        scratch_types=[
            pltpu.VMEM((bpw,), jnp.int32),
            pltpu.VMEM((bpw, dp), jnp.float32),
            pltpu.SemaphoreType.DMA,
        ],
    )
    def k(table_hbm, idx_hbm, out_hbm, idx_v, rows_v, sem):
        wid = lax.axis_index("s") * nc + lax.axis_index("c")
        base = wid * bpw
        pltpu.sync_copy(idx_hbm.at[pl.ds(base, bpw)], idx_v)
        pltpu.async_copy(table_hbm.at[idx_v], rows_v, sem).wait()
        pltpu.sync_copy(rows_v, out_hbm.at[pl.ds(base, bpw)])

    return k(tablep, idx)[:, :_D]


def _conv3(h, w0, w1, w2, b):
    zrow = jnp.zeros((1, h.shape[1]), h.dtype)
    prev = jnp.concatenate([zrow, h[:-1]], axis=0)
    nxt = jnp.concatenate([h[1:], zrow], axis=0)
    y = _mm(prev, w0) + _mm(h, w1) + _mm(nxt, w2) + b
    return jnp.maximum(y, 0.0)


def _attn(h, wq, bq, wk, bk, wv, bv, wo, bo):
    q = _mm(h, wq) + bq
    k = _mm(h, wk) + bk
    v = _mm(h, wv) + bv
    s = _mm_t(q, k) * (1.0 / 8.0)
    m = jnp.max(s, axis=1, keepdims=True)
    e = jnp.exp(s - m)
    a = e / jnp.sum(e, axis=1, keepdims=True)
    o = _mm(_mm(a, v), wo) + bo
    return h + o


def _dec_body(idx_sref, emb_ref,
              a0wq, a0bq, a0wk, a0bk, a0wv, a0bv, a0wo, a0bo,
              a1wq, a1bq, a1wk, a1bk, a1wv, a1bv, a1wo, a1bo,
              d1w0, d1w1, d1w2, d1b,
              d2w0, d2w1, d2w2, d2b,
              d3w0, d3w1, d3w2, d3b,
              rw_ref, rb_ref,
              out_ref, zqt_ref, zq_scr):
    for sb in range(2):
        base = (pl.program_id(0) * 2 + sb) * _T

        def g(i, c, base=base, sb=sb):
            t0 = i * 8
            rows = [emb_ref[pl.ds(idx_sref[base + t0 + u], 1), :]
                    for u in range(8)]
            zq_scr[pl.ds(sb * _T + t0, 8), :] = jnp.concatenate(rows, axis=0)
            return c

        lax.fori_loop(0, _T // 8, g, 0)

    for sb in range(2):
        h = zq_scr[pl.ds(sb * _T, _T), :]
        zqt_ref[sb] = h.T
        h = _attn(h, a0wq[...], a0bq[...], a0wk[...], a0bk[...],
                  a0wv[...], a0bv[...], a0wo[...], a0bo[...])
        h = _attn(h, a1wq[...], a1bq[...], a1wk[...], a1bk[...],
                  a1wv[...], a1bv[...], a1wo[...], a1bo[...])
        h = _conv3(h, d1w0[...], d1w1[...], d1w2[...], d1b[...])
        h = _conv3(h, d2w0[...], d2w1[...], d2w2[...], d2b[...])
        h = _conv3(h, d3w0[...], d3w1[...], d3w2[...], d3b[...])
        out_ref[sb] = (_mm(h, rw_ref[...]) + rb_ref[...]).T


def _full_spec(arr):
    nd = arr.ndim
    return pl.BlockSpec(arr.shape, lambda b, *_, _nd=nd: (0,) * _nd)


def _attn_flat(a):
    return [a['wq'], a['bq'].reshape(1, -1), a['wk'], a['bk'].reshape(1, -1),
            a['wv'], a['bv'].reshape(1, -1), a['wo'], a['bo'].reshape(1, -1)]


def _conv_flat(wb):
    w, b = wb
    return [w[:, :, 0].T, w[:, :, 1].T, w[:, :, 2].T, b.reshape(1, -1)]


def kernel(x, params):
    p = params
    emb = p['emb']

    z = _encode(x, p)
    zt = jnp.transpose(z, (0, 2, 1))
    e2 = jnp.sum(emb**2, axis=1)[None, :]

    idx2 = pl.pallas_call(
        _vq_body,
        grid=(1,),
        in_specs=[pl.BlockSpec((_NT, _D), lambda b, *_: (0, 0)),
                  _full_spec(emb), _full_spec(e2)],
        out_specs=pl.BlockSpec((_NT, 1), lambda b, *_: (0, 0)),
        out_shape=jax.ShapeDtypeStruct((_NT, 1), jnp.int32),
    )(zt.reshape(_NT, _D), emb, e2)

    indices = idx2.reshape(_B, _T)
    idx_flat = idx2.reshape(-1)

    dec_in = [emb]
    for a in p['dec_attn']:
        dec_in += _attn_flat(a)
    for wb in p['dec_conv']:
        dec_in += _conv_flat(wb)
    dec_in += [p['rw'], p['rb'].reshape(1, -1)]

    recon, zq = pl.pallas_call(
        _dec_body,
        grid_spec=pltpu.PrefetchScalarGridSpec(
            num_scalar_prefetch=1,
            grid=(_B // 2,),
            in_specs=[_full_spec(a) for a in dec_in],
            out_specs=[pl.BlockSpec((2, _NB, _T), lambda b, *_: (b, 0, 0)),
                       pl.BlockSpec((2, _D, _T), lambda b, *_: (b, 0, 0))],
            scratch_shapes=[pltpu.VMEM((2 * _T, _D), jnp.float32)],
        ),
        out_shape=[jax.ShapeDtypeStruct((_B, _NB, _T), jnp.float32),
                   jax.ShapeDtypeStruct((_B, _D, _T), jnp.float32)],
    )(idx_flat, *dec_in)

    return recon, z, zq, indices


# speedup vs baseline: 1.6321x; 1.0557x over previous
import functools

import jax
import jax.numpy as jnp
from jax import lax
from jax.experimental import pallas as pl
from jax.experimental.pallas import tpu as pltpu
from jax.experimental.pallas import tpu_sc as plsc

_B, _NB, _T, _D, _K = 4, 96, 256, 64, 8192
_KC = 2048


def _mm(a, b):
    return lax.dot_general(a, b, (((1,), (0,)), ((), ())),
                           preferred_element_type=jnp.float32)


def _mm_t(a, b):
    return lax.dot_general(a, b, (((1,), (1,)), ((), ())),
                           preferred_element_type=jnp.float32)


def _conv1d(x, w, b):
    y = lax.conv_general_dilated(x, w, window_strides=(1,), padding='SAME',
                                 dimension_numbers=('NCH', 'OIH', 'NCH'))
    return y + b[None, :, None]


def _attn_blk(x, a):
    xt = jnp.transpose(x, (0, 2, 1))
    q = xt @ a['wq'] + a['bq']
    k = xt @ a['wk'] + a['bk']
    v = xt @ a['wv'] + a['bv']
    scale = jnp.sqrt(jnp.asarray(q.shape[-1], dtype=x.dtype))
    attn = jax.nn.softmax(q @ jnp.transpose(k, (0, 2, 1)) / scale, axis=-1)
    o = (attn @ v) @ a['wo'] + a['bo']
    return x + jnp.transpose(o, (0, 2, 1))


def _encode(x, p):
    z = x * p['w_proj'][None, :, None]
    for w, b in p['enc_conv']:
        z = jax.nn.relu(_conv1d(z, w, b))
    for a in p['enc_attn']:
        z = _attn_blk(z, a)
    return z


_NT = _B * _T


def _vq_body(z_ref, emb_ref, e2_ref, idx_ref):
    f = jnp.concatenate([z_ref[b].T for b in range(_B)], axis=0)
    f2 = jnp.sum(f * f, axis=1, keepdims=True)
    ids = lax.broadcasted_iota(jnp.int32, (_NT, _KC), 1)

    bd = jnp.full((_NT, 1), jnp.inf, jnp.float32)
    bi = jnp.zeros((_NT, 1), jnp.int32)
    for j in range(_K // _KC):
        e = emb_ref[j * _KC:(j + 1) * _KC, :]
        g2 = _mm_t(f, e * -2.0)
        e2 = e2_ref[0, j * _KC:(j + 1) * _KC][None, :]
        d = (f2 + g2) + e2
        dmin = jnp.min(d, axis=1, keepdims=True)
        imin = jnp.min(jnp.where(d == dmin, ids, jnp.int32(2**31 - 1)),
                       axis=1, keepdims=True) + j * _KC
        better = dmin < bd
        bd = jnp.where(better, dmin, bd)
        bi = jnp.where(better, imin, bi)
    idx_ref[...] = bi


def _gather_sc(table, idx):
    dp = 128
    tablep = jnp.pad(table, ((0, 0), (0, dp - table.shape[1])))
    info = plsc.get_sparse_core_info()
    nw = info.num_cores * info.num_subcores
    bt = idx.shape[0]
    bpw = bt // nw
    nc = info.num_cores
    mesh = plsc.VectorSubcoreMesh(core_axis_name="c", subcore_axis_name="s")

    @functools.partial(
        pl.kernel, mesh=mesh,
        out_type=jax.ShapeDtypeStruct((bt, dp), jnp.float32),
        scratch_types=[
            pltpu.VMEM((bpw,), jnp.int32),
            pltpu.VMEM((bpw, dp), jnp.float32),
            pltpu.SemaphoreType.DMA,
        ],
    )
    def k(table_hbm, idx_hbm, out_hbm, idx_v, rows_v, sem):
        wid = lax.axis_index("s") * nc + lax.axis_index("c")
        base = wid * bpw
        pltpu.sync_copy(idx_hbm.at[pl.ds(base, bpw)], idx_v)
        pltpu.async_copy(table_hbm.at[idx_v], rows_v, sem).wait()
        pltpu.sync_copy(rows_v, out_hbm.at[pl.ds(base, bpw)])

    return k(tablep, idx)[:, :_D]


def _conv3(h, w0, w1, w2, b):
    zrow = jnp.zeros((1, h.shape[1]), h.dtype)
    prev = jnp.concatenate([zrow, h[:-1]], axis=0)
    nxt = jnp.concatenate([h[1:], zrow], axis=0)
    y = _mm(prev, w0) + _mm(h, w1) + _mm(nxt, w2) + b
    return jnp.maximum(y, 0.0)


def _attn(h, wq, bq, wk, bk, wv, bv, wo, bo):
    q = _mm(h, wq) + bq
    k = _mm(h, wk) + bk
    v = _mm(h, wv) + bv
    s = _mm_t(q, k) * (1.0 / 8.0)
    m = jnp.max(s, axis=1, keepdims=True)
    e = jnp.exp(s - m)
    a = e / jnp.sum(e, axis=1, keepdims=True)
    o = _mm(_mm(a, v), wo) + bo
    return h + o


def _dec_body(idx_sref, emb_ref,
              a0wq, a0bq, a0wk, a0bk, a0wv, a0bv, a0wo, a0bo,
              a1wq, a1bq, a1wk, a1bk, a1wv, a1bv, a1wo, a1bo,
              d1w0, d1w1, d1w2, d1b,
              d2w0, d2w1, d2w2, d2b,
              d3w0, d3w1, d3w2, d3b,
              rw_ref, rb_ref,
              out_ref, zqt_ref, zq_scr):
    for sb in range(2):
        base = (pl.program_id(0) * 2 + sb) * _T

        def g(i, c, base=base, sb=sb):
            t0 = i * 8
            rows = [emb_ref[pl.ds(idx_sref[base + t0 + u], 1), :]
                    for u in range(8)]
            zq_scr[pl.ds(sb * _T + t0, 8), :] = jnp.concatenate(rows, axis=0)
            return c

        lax.fori_loop(0, _T // 8, g, 0)

    for sb in range(2):
        h = zq_scr[pl.ds(sb * _T, _T), :]
        zqt_ref[sb] = h.T
        h = _attn(h, a0wq[...], a0bq[...], a0wk[...], a0bk[...],
                  a0wv[...], a0bv[...], a0wo[...], a0bo[...])
        h = _attn(h, a1wq[...], a1bq[...], a1wk[...], a1bk[...],
                  a1wv[...], a1bv[...], a1wo[...], a1bo[...])
        h = _conv3(h, d1w0[...], d1w1[...], d1w2[...], d1b[...])
        h = _conv3(h, d2w0[...], d2w1[...], d2w2[...], d2b[...])
        h = _conv3(h, d3w0[...], d3w1[...], d3w2[...], d3b[...])
        out_ref[sb] = (_mm(h, rw_ref[...]) + rb_ref[...]).T


def _full_spec(arr):
    nd = arr.ndim
    return pl.BlockSpec(arr.shape, lambda b, *_, _nd=nd: (0,) * _nd)


def _attn_flat(a):
    return [a['wq'], a['bq'].reshape(1, -1), a['wk'], a['bk'].reshape(1, -1),
            a['wv'], a['bv'].reshape(1, -1), a['wo'], a['bo'].reshape(1, -1)]


def _conv_flat(wb):
    w, b = wb
    return [w[:, :, 0].T, w[:, :, 1].T, w[:, :, 2].T, b.reshape(1, -1)]


def kernel(x, params):
    p = params
    emb = p['emb']

    z = _encode(x, p)
    e2 = jnp.sum(emb**2, axis=1)[None, :]

    idx2 = pl.pallas_call(
        _vq_body,
        grid=(1,),
        in_specs=[pl.BlockSpec((_B, _D, _T), lambda b, *_: (0, 0, 0)),
                  _full_spec(emb), _full_spec(e2)],
        out_specs=pl.BlockSpec((_NT, 1), lambda b, *_: (0, 0)),
        out_shape=jax.ShapeDtypeStruct((_NT, 1), jnp.int32),
    )(z, emb, e2)

    indices = idx2.reshape(_B, _T)
    idx_flat = idx2.reshape(-1)

    dec_in = [emb]
    for a in p['dec_attn']:
        dec_in += _attn_flat(a)
    for wb in p['dec_conv']:
        dec_in += _conv_flat(wb)
    dec_in += [p['rw'], p['rb'].reshape(1, -1)]

    recon, zq = pl.pallas_call(
        _dec_body,
        grid_spec=pltpu.PrefetchScalarGridSpec(
            num_scalar_prefetch=1,
            grid=(_B // 2,),
            in_specs=[_full_spec(a) for a in dec_in],
            out_specs=[pl.BlockSpec((2, _NB, _T), lambda b, *_: (b, 0, 0)),
                       pl.BlockSpec((2, _D, _T), lambda b, *_: (b, 0, 0))],
            scratch_shapes=[pltpu.VMEM((2 * _T, _D), jnp.float32)],
        ),
        out_shape=[jax.ShapeDtypeStruct((_B, _NB, _T), jnp.float32),
                   jax.ShapeDtypeStruct((_B, _D, _T), jnp.float32)],
    )(idx_flat, *dec_in)

    return recon, z, zq, indices


# speedup vs baseline: 1.7385x; 1.0652x over previous
import functools

import jax
import jax.numpy as jnp
from jax import lax
from jax.experimental import pallas as pl
from jax.experimental.pallas import tpu as pltpu
from jax.experimental.pallas import tpu_sc as plsc

_B, _NB, _T, _D, _K = 4, 96, 256, 64, 8192
_KC = 2048


def _mm(a, b):
    return lax.dot_general(a, b, (((1,), (0,)), ((), ())),
                           preferred_element_type=jnp.float32)


def _mm_t(a, b):
    return lax.dot_general(a, b, (((1,), (1,)), ((), ())),
                           preferred_element_type=jnp.float32)


def _conv1d(x, w, b):
    y = lax.conv_general_dilated(x, w, window_strides=(1,), padding='SAME',
                                 dimension_numbers=('NCH', 'OIH', 'NCH'))
    return y + b[None, :, None]


def _attn_blk(x, a):
    xt = jnp.transpose(x, (0, 2, 1))
    q = xt @ a['wq'] + a['bq']
    k = xt @ a['wk'] + a['bk']
    v = xt @ a['wv'] + a['bv']
    scale = jnp.sqrt(jnp.asarray(q.shape[-1], dtype=x.dtype))
    attn = jax.nn.softmax(q @ jnp.transpose(k, (0, 2, 1)) / scale, axis=-1)
    o = (attn @ v) @ a['wo'] + a['bo']
    return x + jnp.transpose(o, (0, 2, 1))


def _encode(x, p):
    z = x * p['w_proj'][None, :, None]
    for w, b in p['enc_conv']:
        z = jax.nn.relu(_conv1d(z, w, b))
    for a in p['enc_attn']:
        z = _attn_blk(z, a)
    return z


_NT = _B * _T


def _vq_body(z_ref, emb_ref, e2_ref, idx_ref):
    f = jnp.concatenate([z_ref[b].T for b in range(_B)], axis=0)
    f2 = jnp.sum(f * f, axis=1, keepdims=True)
    ids = lax.broadcasted_iota(jnp.int32, (_NT, _KC), 1)

    bd = jnp.full((_NT, 1), jnp.inf, jnp.float32)
    bi = jnp.zeros((_NT, 1), jnp.int32)
    for j in range(_K // _KC):
        e = emb_ref[j * _KC:(j + 1) * _KC, :]
        g2 = _mm_t(f, e * -2.0)
        e2 = e2_ref[0, j * _KC:(j + 1) * _KC][None, :]
        d = (f2 + g2) + e2
        dmin = jnp.min(d, axis=1, keepdims=True)
        imin = jnp.min(jnp.where(d == dmin, ids, jnp.int32(2**31 - 1)),
                       axis=1, keepdims=True) + j * _KC
        better = dmin < bd
        bd = jnp.where(better, dmin, bd)
        bi = jnp.where(better, imin, bi)
    idx_ref[...] = bi


def _gather_sc(table, idx):
    dp = 128
    tablep = jnp.pad(table, ((0, 0), (0, dp - table.shape[1])))
    info = plsc.get_sparse_core_info()
    nw = info.num_cores * info.num_subcores
    bt = idx.shape[0]
    bpw = bt // nw
    nc = info.num_cores
    mesh = plsc.VectorSubcoreMesh(core_axis_name="c", subcore_axis_name="s")

    @functools.partial(
        pl.kernel, mesh=mesh,
        out_type=jax.ShapeDtypeStruct((bt, dp), jnp.float32),
        scratch_types=[
            pltpu.VMEM((bpw,), jnp.int32),
            pltpu.VMEM((bpw, dp), jnp.float32),
            pltpu.SemaphoreType.DMA,
        ],
    )
    def k(table_hbm, idx_hbm, out_hbm, idx_v, rows_v, sem):
        wid = lax.axis_index("s") * nc + lax.axis_index("c")
        base = wid * bpw
        pltpu.sync_copy(idx_hbm.at[pl.ds(base, bpw)], idx_v)
        pltpu.async_copy(table_hbm.at[idx_v], rows_v, sem).wait()
        pltpu.sync_copy(rows_v, out_hbm.at[pl.ds(base, bpw)])

    return k(tablep, idx)[:, :_D]


def _conv3_pair(hs, w0, w1, w2, b):
    zrow = jnp.zeros((1, hs[0].shape[1]), hs[0].dtype)
    prev = [jnp.concatenate([zrow, h[:-1]], axis=0) for h in hs]
    nxt = [jnp.concatenate([h[1:], zrow], axis=0) for h in hs]
    y = [_mm(p, w0) + _mm(h, w1) + _mm(n, w2) + b
         for p, h, n in zip(prev, hs, nxt)]
    return [jnp.maximum(yi, 0.0) for yi in y]


def _attn_pair(hs, wq, bq, wk, bk, wv, bv, wo, bo):
    q = [_mm(h, wq) + bq for h in hs]
    k = [_mm(h, wk) + bk for h in hs]
    v = [_mm(h, wv) + bv for h in hs]
    s = [_mm_t(qi, ki) * (1.0 / 8.0) for qi, ki in zip(q, k)]
    m = [jnp.max(si, axis=1, keepdims=True) for si in s]
    e = [jnp.exp(si - mi) for si, mi in zip(s, m)]
    a = [ei / jnp.sum(ei, axis=1, keepdims=True) for ei in e]
    av = [_mm(ai, vi) for ai, vi in zip(a, v)]
    return [h + (_mm(avi, wo) + bo) for h, avi in zip(hs, av)]


def _dec_body(idx_sref, emb_ref,
              a0wq, a0bq, a0wk, a0bk, a0wv, a0bv, a0wo, a0bo,
              a1wq, a1bq, a1wk, a1bk, a1wv, a1bv, a1wo, a1bo,
              d1w0, d1w1, d1w2, d1b,
              d2w0, d2w1, d2w2, d2b,
              d3w0, d3w1, d3w2, d3b,
              rw_ref, rb_ref,
              out_ref, zqt_ref, zq_scr):
    for sb in range(2):
        base = (pl.program_id(0) * 2 + sb) * _T

        def g(i, c, base=base, sb=sb):
            t0 = i * 8
            rows = [emb_ref[pl.ds(idx_sref[base + t0 + u], 1), :]
                    for u in range(8)]
            zq_scr[pl.ds(sb * _T + t0, 8), :] = jnp.concatenate(rows, axis=0)
            return c

        lax.fori_loop(0, _T // 8, g, 0)

    hs = [zq_scr[pl.ds(sb * _T, _T), :] for sb in range(2)]
    for sb in range(2):
        zqt_ref[sb] = hs[sb].T
    hs = _attn_pair(hs, a0wq[...], a0bq[...], a0wk[...], a0bk[...],
                    a0wv[...], a0bv[...], a0wo[...], a0bo[...])
    hs = _attn_pair(hs, a1wq[...], a1bq[...], a1wk[...], a1bk[...],
                    a1wv[...], a1bv[...], a1wo[...], a1bo[...])
    hs = _conv3_pair(hs, d1w0[...], d1w1[...], d1w2[...], d1b[...])
    hs = _conv3_pair(hs, d2w0[...], d2w1[...], d2w2[...], d2b[...])
    hs = _conv3_pair(hs, d3w0[...], d3w1[...], d3w2[...], d3b[...])
    ys = [_mm(h, rw_ref[...]) + rb_ref[...] for h in hs]
    for sb in range(2):
        out_ref[sb] = ys[sb].T


def _full_spec(arr):
    nd = arr.ndim
    return pl.BlockSpec(arr.shape, lambda b, *_, _nd=nd: (0,) * _nd)


def _attn_flat(a):
    return [a['wq'], a['bq'].reshape(1, -1), a['wk'], a['bk'].reshape(1, -1),
            a['wv'], a['bv'].reshape(1, -1), a['wo'], a['bo'].reshape(1, -1)]


def _conv_flat(wb):
    w, b = wb
    return [w[:, :, 0].T, w[:, :, 1].T, w[:, :, 2].T, b.reshape(1, -1)]


def kernel(x, params):
    p = params
    emb = p['emb']

    z = _encode(x, p)
    e2 = jnp.sum(emb**2, axis=1)[None, :]

    idx2 = pl.pallas_call(
        _vq_body,
        grid=(1,),
        in_specs=[pl.BlockSpec((_B, _D, _T), lambda b, *_: (0, 0, 0)),
                  _full_spec(emb), _full_spec(e2)],
        out_specs=pl.BlockSpec((_NT, 1), lambda b, *_: (0, 0)),
        out_shape=jax.ShapeDtypeStruct((_NT, 1), jnp.int32),
    )(z, emb, e2)

    indices = idx2.reshape(_B, _T)
    idx_flat = idx2.reshape(-1)

    dec_in = [emb]
    for a in p['dec_attn']:
        dec_in += _attn_flat(a)
    for wb in p['dec_conv']:
        dec_in += _conv_flat(wb)
    dec_in += [p['rw'], p['rb'].reshape(1, -1)]

    recon, zq = pl.pallas_call(
        _dec_body,
        grid_spec=pltpu.PrefetchScalarGridSpec(
            num_scalar_prefetch=1,
            grid=(_B // 2,),
            in_specs=[_full_spec(a) for a in dec_in],
            out_specs=[pl.BlockSpec((2, _NB, _T), lambda b, *_: (b, 0, 0)),
                       pl.BlockSpec((2, _D, _T), lambda b, *_: (b, 0, 0))],
            scratch_shapes=[pltpu.VMEM((2 * _T, _D), jnp.float32)],
        ),
        out_shape=[jax.ShapeDtypeStruct((_B, _NB, _T), jnp.float32),
                   jax.ShapeDtypeStruct((_B, _D, _T), jnp.float32)],
    )(idx_flat, *dec_in)

    return recon, z, zq, indices


# speedup vs baseline: 1.8007x; 1.0358x over previous
import functools

import jax
import jax.numpy as jnp
from jax import lax
from jax.experimental import pallas as pl
from jax.experimental.pallas import tpu as pltpu
from jax.experimental.pallas import tpu_sc as plsc

_B, _NB, _T, _D, _K = 4, 96, 256, 64, 8192
_KC = 2048


def _mm(a, b):
    return lax.dot_general(a, b, (((1,), (0,)), ((), ())),
                           preferred_element_type=jnp.float32)


def _mm_t(a, b):
    return lax.dot_general(a, b, (((1,), (1,)), ((), ())),
                           preferred_element_type=jnp.float32)


def _conv1d(x, w, b):
    y = lax.conv_general_dilated(x, w, window_strides=(1,), padding='SAME',
                                 dimension_numbers=('NCH', 'OIH', 'NCH'))
    return y + b[None, :, None]


def _attn_blk(x, a):
    xt = jnp.transpose(x, (0, 2, 1))
    q = xt @ a['wq'] + a['bq']
    k = xt @ a['wk'] + a['bk']
    v = xt @ a['wv'] + a['bv']
    scale = jnp.sqrt(jnp.asarray(q.shape[-1], dtype=x.dtype))
    attn = jax.nn.softmax(q @ jnp.transpose(k, (0, 2, 1)) / scale, axis=-1)
    o = (attn @ v) @ a['wo'] + a['bo']
    return x + jnp.transpose(o, (0, 2, 1))


def _encode(x, p):
    z = x * p['w_proj'][None, :, None]
    for w, b in p['enc_conv']:
        z = jax.nn.relu(_conv1d(z, w, b))
    for a in p['enc_attn']:
        z = _attn_blk(z, a)
    return z


_NT = _B * _T


def _vq_body(z_ref, emb_ref, e2_ref, idx_ref):
    f = jnp.concatenate([z_ref[b].T for b in range(_B)], axis=0)
    f2 = jnp.sum(f * f, axis=1, keepdims=True)
    ids = lax.broadcasted_iota(jnp.int32, (1, _KC), 1)

    bd = jnp.full((_NT, 1), jnp.inf, jnp.float32)
    bi = jnp.zeros((_NT, 1), jnp.int32)
    for j in range(_K // _KC):
        e = emb_ref[j * _KC:(j + 1) * _KC, :]
        g2 = _mm_t(f, e * -2.0)
        e2 = e2_ref[0, j * _KC:(j + 1) * _KC][None, :]
        d = (f2 + g2) + e2
        dmin = jnp.min(d, axis=1, keepdims=True)
        imin = jnp.min(jnp.where(d == dmin, ids, jnp.int32(2**31 - 1)),
                       axis=1, keepdims=True) + j * _KC
        better = dmin < bd
        bd = jnp.where(better, dmin, bd)
        bi = jnp.where(better, imin, bi)
    idx_ref[...] = bi


def _gather_sc(table, idx):
    dp = 128
    tablep = jnp.pad(table, ((0, 0), (0, dp - table.shape[1])))
    info = plsc.get_sparse_core_info()
    nw = info.num_cores * info.num_subcores
    bt = idx.shape[0]
    bpw = bt // nw
    nc = info.num_cores
    mesh = plsc.VectorSubcoreMesh(core_axis_name="c", subcore_axis_name="s")

    @functools.partial(
        pl.kernel, mesh=mesh,
        out_type=jax.ShapeDtypeStruct((bt, dp), jnp.float32),
        scratch_types=[
            pltpu.VMEM((bpw,), jnp.int32),
            pltpu.VMEM((bpw, dp), jnp.float32),
            pltpu.SemaphoreType.DMA,
        ],
    )
    def k(table_hbm, idx_hbm, out_hbm, idx_v, rows_v, sem):
        wid = lax.axis_index("s") * nc + lax.axis_index("c")
        base = wid * bpw
        pltpu.sync_copy(idx_hbm.at[pl.ds(base, bpw)], idx_v)
        pltpu.async_copy(table_hbm.at[idx_v], rows_v, sem).wait()
        pltpu.sync_copy(rows_v, out_hbm.at[pl.ds(base, bpw)])

    return k(tablep, idx)[:, :_D]


def _conv3_pair(hs, w0, w1, w2, b):
    zrow = jnp.zeros((1, hs[0].shape[1]), hs[0].dtype)
    prev = [jnp.concatenate([zrow, h[:-1]], axis=0) for h in hs]
    nxt = [jnp.concatenate([h[1:], zrow], axis=0) for h in hs]
    y = [_mm(p, w0) + _mm(h, w1) + _mm(n, w2) + b
         for p, h, n in zip(prev, hs, nxt)]
    return [jnp.maximum(yi, 0.0) for yi in y]


def _attn_pair(hs, wq, bq, wk, bk, wv, bv, wo, bo):
    q = [_mm(h, wq) + bq for h in hs]
    k = [_mm(h, wk) + bk for h in hs]
    v = [_mm(h, wv) + bv for h in hs]
    s = [_mm_t(qi, ki) * (1.0 / 8.0) for qi, ki in zip(q, k)]
    m = [jnp.max(si, axis=1, keepdims=True) for si in s]
    e = [jnp.exp(si - mi) for si, mi in zip(s, m)]
    a = [ei / jnp.sum(ei, axis=1, keepdims=True) for ei in e]
    av = [_mm(ai, vi) for ai, vi in zip(a, v)]
    return [h + (_mm(avi, wo) + bo) for h, avi in zip(hs, av)]


def _dec_body(idx_sref, emb_ref,
              a0wq, a0bq, a0wk, a0bk, a0wv, a0bv, a0wo, a0bo,
              a1wq, a1bq, a1wk, a1bk, a1wv, a1bv, a1wo, a1bo,
              d1w0, d1w1, d1w2, d1b,
              d2w0, d2w1, d2w2, d2b,
              d3w0, d3w1, d3w2, d3b,
              rw_ref, rb_ref,
              out_ref, zqt_ref, zq_scr):
    def g(i, c):
        t0 = i * 8
        rows = [emb_ref[pl.ds(idx_sref[t0 + u], 1), :] for u in range(8)]
        zq_scr[pl.ds(t0, 8), :] = jnp.concatenate(rows, axis=0)
        return c

    lax.fori_loop(0, _NT // 8, g, 0)

    hs = [zq_scr[pl.ds(sb * _T, _T), :] for sb in range(_B)]
    for sb in range(_B):
        zqt_ref[sb] = hs[sb].T
    hs = _attn_pair(hs, a0wq[...], a0bq[...], a0wk[...], a0bk[...],
                    a0wv[...], a0bv[...], a0wo[...], a0bo[...])
    hs = _attn_pair(hs, a1wq[...], a1bq[...], a1wk[...], a1bk[...],
                    a1wv[...], a1bv[...], a1wo[...], a1bo[...])
    hs = _conv3_pair(hs, d1w0[...], d1w1[...], d1w2[...], d1b[...])
    hs = _conv3_pair(hs, d2w0[...], d2w1[...], d2w2[...], d2b[...])
    hs = _conv3_pair(hs, d3w0[...], d3w1[...], d3w2[...], d3b[...])
    ys = [_mm(h, rw_ref[...]) + rb_ref[...] for h in hs]
    for sb in range(_B):
        out_ref[sb] = ys[sb].T


def _full_spec(arr):
    nd = arr.ndim
    return pl.BlockSpec(arr.shape, lambda b, *_, _nd=nd: (0,) * _nd)


def _attn_flat(a):
    return [a['wq'], a['bq'].reshape(1, -1), a['wk'], a['bk'].reshape(1, -1),
            a['wv'], a['bv'].reshape(1, -1), a['wo'], a['bo'].reshape(1, -1)]


def _conv_flat(wb):
    w, b = wb
    return [w[:, :, 0].T, w[:, :, 1].T, w[:, :, 2].T, b.reshape(1, -1)]


def kernel(x, params):
    p = params
    emb = p['emb']

    z = _encode(x, p)
    e2 = jnp.sum(emb**2, axis=1)[None, :]

    idx2 = pl.pallas_call(
        _vq_body,
        grid=(1,),
        in_specs=[pl.BlockSpec((_B, _D, _T), lambda b, *_: (0, 0, 0)),
                  _full_spec(emb), _full_spec(e2)],
        out_specs=pl.BlockSpec((_NT, 1), lambda b, *_: (0, 0)),
        out_shape=jax.ShapeDtypeStruct((_NT, 1), jnp.int32),
    )(z, emb, e2)

    indices = idx2.reshape(_B, _T)
    idx_flat = idx2.reshape(-1)

    dec_in = [emb]
    for a in p['dec_attn']:
        dec_in += _attn_flat(a)
    for wb in p['dec_conv']:
        dec_in += _conv_flat(wb)
    dec_in += [p['rw'], p['rb'].reshape(1, -1)]

    recon, zq = pl.pallas_call(
        _dec_body,
        grid_spec=pltpu.PrefetchScalarGridSpec(
            num_scalar_prefetch=1,
            grid=(1,),
            in_specs=[_full_spec(a) for a in dec_in],
            out_specs=[pl.BlockSpec((_B, _NB, _T), lambda b, *_: (0, 0, 0)),
                       pl.BlockSpec((_B, _D, _T), lambda b, *_: (0, 0, 0))],
            scratch_shapes=[pltpu.VMEM((_NT, _D), jnp.float32)],
        ),
        out_shape=[jax.ShapeDtypeStruct((_B, _NB, _T), jnp.float32),
                   jax.ShapeDtypeStruct((_B, _D, _T), jnp.float32)],
    )(idx_flat, *dec_in)

    return recon, z, zq, indices


# speedup vs baseline: 1.8027x; 1.0011x over previous
import jax
import jax.numpy as jnp
from jax import lax
from jax.experimental import pallas as pl
from jax.experimental.pallas import tpu as pltpu

_B, _NB, _T, _D, _K = 4, 96, 256, 64, 8192
_KC = 2048


def _mm(a, b):
    return lax.dot_general(a, b, (((1,), (0,)), ((), ())),
                           preferred_element_type=jnp.float32)


def _mm_t(a, b):
    return lax.dot_general(a, b, (((1,), (1,)), ((), ())),
                           preferred_element_type=jnp.float32)


def _conv1d(x, w, b):
    y = lax.conv_general_dilated(x, w, window_strides=(1,), padding='SAME',
                                 dimension_numbers=('NCH', 'OIH', 'NCH'))
    return y + b[None, :, None]


def _attn_blk(x, a):
    xt = jnp.transpose(x, (0, 2, 1))
    q = xt @ a['wq'] + a['bq']
    k = xt @ a['wk'] + a['bk']
    v = xt @ a['wv'] + a['bv']
    scale = jnp.sqrt(jnp.asarray(q.shape[-1], dtype=x.dtype))
    attn = jax.nn.softmax(q @ jnp.transpose(k, (0, 2, 1)) / scale, axis=-1)
    o = (attn @ v) @ a['wo'] + a['bo']
    return x + jnp.transpose(o, (0, 2, 1))


def _encode(x, p):
    z = x * p['w_proj'][None, :, None]
    for w, b in p['enc_conv']:
        z = jax.nn.relu(_conv1d(z, w, b))
    for a in p['enc_attn']:
        z = _attn_blk(z, a)
    return z


_NT = _B * _T


def _vq_body(z_ref, emb_ref, e2_ref, idx_ref):
    f = jnp.concatenate([z_ref[b].T for b in range(_B)], axis=0)
    f2 = jnp.sum(f * f, axis=1, keepdims=True)
    ids = lax.broadcasted_iota(jnp.int32, (1, _KC), 1)

    bd = jnp.full((_NT, 1), jnp.inf, jnp.float32)
    bi = jnp.zeros((_NT, 1), jnp.int32)
    for j in range(_K // _KC):
        e = emb_ref[j * _KC:(j + 1) * _KC, :]
        g2 = _mm_t(f, e * -2.0)
        e2 = e2_ref[0, j * _KC:(j + 1) * _KC][None, :]
        d = (f2 + g2) + e2
        dmin = jnp.min(d, axis=1, keepdims=True)
        imin = jnp.min(jnp.where(d == dmin, ids, jnp.int32(2**31 - 1)),
                       axis=1, keepdims=True) + j * _KC
        better = dmin < bd
        bd = jnp.where(better, dmin, bd)
        bi = jnp.where(better, imin, bi)
    idx_ref[...] = bi


def _conv3_pair(hs, w0, w1, w2, b):
    zrow = jnp.zeros((1, hs[0].shape[1]), hs[0].dtype)
    prev = [jnp.concatenate([zrow, h[:-1]], axis=0) for h in hs]
    nxt = [jnp.concatenate([h[1:], zrow], axis=0) for h in hs]
    y = [_mm(p, w0) + _mm(h, w1) + _mm(n, w2) + b
         for p, h, n in zip(prev, hs, nxt)]
    return [jnp.maximum(yi, 0.0) for yi in y]


def _attn_pair(hs, wq, bq, wk, bk, wv, bv, wo, bo):
    q = [_mm(h, wq) + bq for h in hs]
    k = [_mm(h, wk) + bk for h in hs]
    v = [_mm(h, wv) + bv for h in hs]
    s = [_mm_t(qi, ki) * (1.0 / 8.0) for qi, ki in zip(q, k)]
    m = [jnp.max(si, axis=1, keepdims=True) for si in s]
    e = [jnp.exp(si - mi) for si, mi in zip(s, m)]
    a = [ei / jnp.sum(ei, axis=1, keepdims=True) for ei in e]
    av = [_mm(ai, vi) for ai, vi in zip(a, v)]
    return [h + (_mm(avi, wo) + bo) for h, avi in zip(hs, av)]


def _dec_body(idx_sref, emb_ref,
              a0wq, a0bq, a0wk, a0bk, a0wv, a0bv, a0wo, a0bo,
              a1wq, a1bq, a1wk, a1bk, a1wv, a1bv, a1wo, a1bo,
              d1w0, d1w1, d1w2, d1b,
              d2w0, d2w1, d2w2, d2b,
              d3w0, d3w1, d3w2, d3b,
              rw_ref, rb_ref,
              out_ref, zqt_ref, zq_scr):
    def g(i, c):
        t0 = i * 8
        rows = [emb_ref[pl.ds(idx_sref[t0 + u], 1), :] for u in range(8)]
        zq_scr[pl.ds(t0, 8), :] = jnp.concatenate(rows, axis=0)
        return c

    lax.fori_loop(0, _NT // 8, g, 0)

    hs = [zq_scr[pl.ds(sb * _T, _T), :] for sb in range(_B)]
    for sb in range(_B):
        zqt_ref[sb] = hs[sb].T
    hs = _attn_pair(hs, a0wq[...], a0bq[...], a0wk[...], a0bk[...],
                    a0wv[...], a0bv[...], a0wo[...], a0bo[...])
    hs = _attn_pair(hs, a1wq[...], a1bq[...], a1wk[...], a1bk[...],
                    a1wv[...], a1bv[...], a1wo[...], a1bo[...])
    hs = _conv3_pair(hs, d1w0[...], d1w1[...], d1w2[...], d1b[...])
    hs = _conv3_pair(hs, d2w0[...], d2w1[...], d2w2[...], d2b[...])
    hs = _conv3_pair(hs, d3w0[...], d3w1[...], d3w2[...], d3b[...])
    ys = [_mm(h, rw_ref[...]) + rb_ref[...] for h in hs]
    for sb in range(_B):
        out_ref[sb] = ys[sb].T


def _full_spec(arr):
    nd = arr.ndim
    return pl.BlockSpec(arr.shape, lambda b, *_, _nd=nd: (0,) * _nd)


def _attn_flat(a):
    return [a['wq'], a['bq'].reshape(1, -1), a['wk'], a['bk'].reshape(1, -1),
            a['wv'], a['bv'].reshape(1, -1), a['wo'], a['bo'].reshape(1, -1)]


def _conv_flat(wb):
    w, b = wb
    return [w[:, :, 0].T, w[:, :, 1].T, w[:, :, 2].T, b.reshape(1, -1)]


def kernel(x, params):
    p = params
    emb = p['emb']

    z = _encode(x, p)
    e2 = jnp.sum(emb**2, axis=1)[None, :]

    idx2 = pl.pallas_call(
        _vq_body,
        grid=(1,),
        in_specs=[pl.BlockSpec((_B, _D, _T), lambda b, *_: (0, 0, 0)),
                  _full_spec(emb), _full_spec(e2)],
        out_specs=pl.BlockSpec((_NT, 1), lambda b, *_: (0, 0)),
        out_shape=jax.ShapeDtypeStruct((_NT, 1), jnp.int32),
    )(z, emb, e2)

    indices = idx2.reshape(_B, _T)
    idx_flat = idx2.reshape(-1)

    dec_in = [emb]
    for a in p['dec_attn']:
        dec_in += _attn_flat(a)
    for wb in p['dec_conv']:
        dec_in += _conv_flat(wb)
    dec_in += [p['rw'], p['rb'].reshape(1, -1)]

    recon, zq = pl.pallas_call(
        _dec_body,
        grid_spec=pltpu.PrefetchScalarGridSpec(
            num_scalar_prefetch=1,
            grid=(1,),
            in_specs=[_full_spec(a) for a in dec_in],
            out_specs=[pl.BlockSpec((_B, _NB, _T), lambda b, *_: (0, 0, 0)),
                       pl.BlockSpec((_B, _D, _T), lambda b, *_: (0, 0, 0))],
            scratch_shapes=[pltpu.VMEM((_NT, _D), jnp.float32)],
        ),
        out_shape=[jax.ShapeDtypeStruct((_B, _NB, _T), jnp.float32),
                   jax.ShapeDtypeStruct((_B, _D, _T), jnp.float32)],
    )(idx_flat, *dec_in)

    return recon, z, zq, indices
